# Initial kernel scaffold; baseline (speedup 1.0000x reference)
import jax
import jax.numpy as jnp
from jax.experimental import pallas as pl


def kernel(xyv, params):
    raise NotImplementedError("write your pallas kernel here")


# speedup vs baseline: 6.8237x; 6.8237x over previous
import functools

import jax
import jax.numpy as jnp
from jax.experimental import pallas as pl
from jax.experimental.pallas import tpu as pltpu

EPS = 1e-5
NEG = -1e30
_INTERPRET = False

_mm = functools.partial(jnp.dot, preferred_element_type=jnp.float32)


def _fps_body(xy_ref, o1_ref, o2_ref, o3_ref):
    def run(pts, S):
        N = pts.shape[1]
        iota_n = jax.lax.broadcasted_iota(jnp.int32, (1, N), 1)
        iota_s = jax.lax.broadcasted_iota(jnp.int32, (1, S), 1)

        def step(i, state):
            dist, far, acc = state
            onehot = (iota_n == far).astype(jnp.float32)
            c = jnp.sum(pts * onehot, axis=1, keepdims=True)
            acc = acc + c * (iota_s == i).astype(jnp.float32)
            d = jnp.sum((pts - c) ** 2, axis=0, keepdims=True)
            dist = jnp.minimum(dist, d)
            far = jnp.argmax(dist, axis=1).reshape(1, 1).astype(jnp.int32)
            return dist, far, acc

        init = (jnp.full((1, N), 1e10, jnp.float32),
                jnp.zeros((1, 1), jnp.int32),
                jnp.zeros((2, S), jnp.float32))
        _, _, acc = jax.lax.fori_loop(0, S, step, init)
        return acc

    p0 = xy_ref[0]
    n1 = run(p0, 1024)
    o1_ref[0] = n1
    n2 = run(n1, 512)
    o2_ref[0] = n2
    n3 = run(n2, 256)
    o3_ref[0] = n3


def _fps_all(xy):
    B, _, N = xy.shape
    outs = [jax.ShapeDtypeStruct((B, 2, s), jnp.float32) for s in (1024, 512, 256)]
    return pl.pallas_call(
        _fps_body,
        grid=(B,),
        in_specs=[pl.BlockSpec((1, 2, N), lambda b: (b, 0, 0))],
        out_specs=[pl.BlockSpec((1, 2, s), lambda b: (b, 0, 0))
                   for s in (1024, 512, 256)],
        out_shape=outs,
        interpret=_INTERPRET,
    )(xy)


def _make_sa_body(N, T, C, branches):
    def body(x2_ref, featx_ref, nxy_ref, *refs):
        wrefs = refs[:len(branches) * 6]
        orefs = refs[len(branches) * 6:]
        x2 = x2_ref[0]
        c = nxy_ref[0]
        s_dst = jnp.sum(x2 * x2, axis=0, keepdims=True)
        s_src = jnp.sum(c * c, axis=1, keepdims=True)
        sqr = s_src + s_dst - 2.0 * _mm(c, x2)
        featx = featx_ref[0]
        for bi, (K, r2) in enumerate(branches):
            match = sqr <= r2
            rank = match.astype(jnp.int32)
            sh = 1
            while sh < N:
                rank = rank + jnp.concatenate(
                    [jnp.zeros((T, sh), jnp.int32), rank[:, :N - sh]], axis=1)
                sh *= 2
            cnt = rank[:, N - 1:N]
            rank_t = jnp.concatenate([rank] * K, axis=0)
            match_t = jnp.concatenate([match] * K, axis=0)
            kk = jax.lax.broadcasted_iota(jnp.int32, (K * T, 1), 0) // T
            sel = jnp.where(jnp.logical_and(match_t, rank_t == kk + 1),
                            1.0, 0.0)
            g = _mm(sel, featx)
            ct = jnp.concatenate([c] * K, axis=0)
            h = jnp.concatenate([g[:, :C], g[:, C:] - ct], axis=1)
            for li in range(3):
                w = wrefs[bi * 6 + 2 * li][...]
                b = wrefs[bi * 6 + 2 * li + 1][...]
                h = jnp.maximum(_mm(h, w) + b, 0.0)
            cnt_t = jnp.concatenate([cnt] * K, axis=0)
            h = jnp.where(kk < cnt_t, h, NEG)
            o = h[0:T, :]
            for k in range(1, K):
                o = jnp.maximum(o, h[k * T:(k + 1) * T, :])
            orefs[bi][0] = o
    return body


def _sa_level(xyz, featx, nxy, branches, weights, T=32):
    B, _, N = xyz.shape
    S = nxy.shape[1]
    C = featx.shape[2] - 2
    in_specs = [
        pl.BlockSpec((1, 2, N), lambda b, t: (b, 0, 0)),
        pl.BlockSpec((1, N, featx.shape[2]), lambda b, t: (b, 0, 0)),
        pl.BlockSpec((1, T, 2), lambda b, t: (b, t, 0)),
    ]
    args = [xyz, featx, nxy]
    for br in weights:
        for (w, b) in br:
            args += [w, b]
            in_specs += [
                pl.BlockSpec(w.shape, lambda bb, tt: (0, 0)),
                pl.BlockSpec(b.shape, lambda bb, tt: (0, 0)),
            ]
    outs = [jax.ShapeDtypeStruct((B, S, br[-1][0].shape[1]), jnp.float32)
            for br in weights]
    out_specs = [pl.BlockSpec((1, T, o.shape[2]), lambda b, t: (b, t, 0))
                 for o in outs]
    return pl.pallas_call(
        _make_sa_body(N, T, C, branches),
        grid=(B, S // T),
        in_specs=in_specs,
        out_specs=out_specs,
        out_shape=outs,
        interpret=_INTERPRET,
    )(*args)


def _make_fp_body(N2, T, has_p1, nlayers):
    def body(*refs):
        x1_ref, x2_ref, p2_ref = refs[0], refs[1], refs[2]
        i = 3
        p1_ref = None
        if has_p1:
            p1_ref = refs[i]
            i += 1
        wrefs = refs[i:i + 2 * nlayers]
        out_ref = refs[i + 2 * nlayers]
        c = x1_ref[0]
        x2 = x2_ref[0]
        s2 = jnp.sum(x2 * x2, axis=0, keepdims=True)
        s1 = jnp.sum(c * c, axis=1, keepdims=True)
        d = s1 + s2 - 2.0 * _mm(c, x2)
        iota_n = jax.lax.broadcasted_iota(jnp.int32, (T, N2), 1)
        dd = d
        dmins, idxs = [], []
        for _ in range(3):
            dmins.append(jnp.min(dd, axis=1, keepdims=True))
            im = jnp.argmin(dd, axis=1).reshape(T, 1).astype(jnp.int32)
            idxs.append(im)
            dd = jnp.where(iota_n == im, 1e30, dd)
        recips = [1.0 / (dm + 1e-8) for dm in dmins]
        norm = recips[0] + recips[1] + recips[2]
        wmat = jnp.zeros((T, N2), jnp.float32)
        for k in range(3):
            wmat = wmat + jnp.where(iota_n == idxs[k], 1.0, 0.0) * (recips[k] / norm)
        h = _mm(wmat, p2_ref[0])
        if has_p1:
            h = jnp.concatenate([p1_ref[0], h], axis=1)
        for li in range(nlayers):
            h = jnp.maximum(_mm(h, wrefs[2 * li][...]) + wrefs[2 * li + 1][...], 0.0)
        out_ref[0] = h
    return body


def _fp_stage(x1s, xyz2, p2, p1, weights, T=256):
    B, N1, _ = x1s.shape
    N2 = xyz2.shape[2]
    in_specs = [
        pl.BlockSpec((1, T, 2), lambda b, t: (b, t, 0)),
        pl.BlockSpec((1, 2, N2), lambda b, t: (b, 0, 0)),
        pl.BlockSpec((1, N2, p2.shape[2]), lambda b, t: (b, 0, 0)),
    ]
    args = [x1s, xyz2, p2]
    if p1 is not None:
        in_specs.append(pl.BlockSpec((1, T, p1.shape[2]), lambda b, t: (b, t, 0)))
        args.append(p1)
    for (w, b) in weights:
        args += [w, b]
        in_specs += [
            pl.BlockSpec(w.shape, lambda bb, tt: (0, 0)),
            pl.BlockSpec(b.shape, lambda bb, tt: (0, 0)),
        ]
    O = weights[-1][0].shape[1]
    out = jax.ShapeDtypeStruct((B, N1, O), jnp.float32)
    return pl.pallas_call(
        _make_fp_body(N2, T, p1 is not None, len(weights)),
        grid=(B, N1 // T),
        in_specs=in_specs,
        out_specs=pl.BlockSpec((1, T, O), lambda b, t: (b, t, 0)),
        out_shape=out,
        interpret=_INTERPRET,
    )(*args)


def _conv_body(x_ref, w1_ref, b1_ref, w2_ref, b2_ref, w3_ref, b3_ref,
               out_ref, xp_ref, acc_ref):
    def stage(xin, w_ref, b, front, L, Lout, relu):
        k = w_ref.shape[0]
        O = w_ref.shape[2]
        xp_ref[...] = jnp.zeros(xp_ref.shape, jnp.float32)
        xp_ref[pl.ds(front, L), :] = xin
        acc_ref[:Lout, :O] = jnp.zeros((Lout, O), jnp.float32)

        def tau(t, _):
            xs = xp_ref[pl.ds(t, Lout), :]
            acc_ref[:Lout, :O] += _mm(xs, w_ref[t])
            return 0

        jax.lax.fori_loop(0, k, tau, 0)
        h = acc_ref[:Lout, :O] + b
        return jnp.maximum(h, 0.0) if relu else h

    x = x_ref[0]
    h1 = stage(x, w1_ref, b1_ref[...], 128, 2048, 2048, True)
    h2 = stage(h1, w2_ref, b2_ref[...], 63, 2048, 2047, True)
    h3 = stage(h2, w3_ref, b3_ref[...], 3, 2047, 2048, False)
    m = jnp.max(h3, axis=1, keepdims=True)
    z = h3 - m
    lse = jnp.log(jnp.sum(jnp.exp(z), axis=1, keepdims=True))
    out_ref[0] = z - lse


def _conv_head(x, w1, b1, w2, b2, w3, b3):
    B = x.shape[0]
    wspecs = []
    for w, b in ((w1, b1), (w2, b2), (w3, b3)):
        wspecs += [pl.BlockSpec(w.shape, lambda bb: (0, 0, 0)),
                   pl.BlockSpec(b.shape, lambda bb: (0, 0))]
    return pl.pallas_call(
        _conv_body,
        grid=(B,),
        in_specs=[pl.BlockSpec((1, 2048, 128), lambda b: (b, 0, 0))] + wspecs,
        out_specs=pl.BlockSpec((1, 2048, 2), lambda b: (b, 0, 0)),
        out_shape=jax.ShapeDtypeStruct((B, 2048, 2), jnp.float32),
        scratch_shapes=[pltpu.VMEM((2304, 128), jnp.float32),
                        pltpu.VMEM((2048, 128), jnp.float32)],
        interpret=_INTERPRET,
    )(x, w1, b1, w2, b2, w3, b3)


def _fold(p):
    s = p['gamma'] / jnp.sqrt(1.0 + EPS)
    return (p['W'] * s[:, None]).T, (p['b'] * s + p['beta'])[None, :]


def _fold_conv(p, bn):
    if bn is None:
        w, b = p['W'], p['b']
    else:
        s = bn['gamma'] / jnp.sqrt(1.0 + EPS)
        w = p['W'] * s[:, None, None]
        b = p['b'] * s + bn['beta']
    return w.transpose(2, 1, 0), b[None, :]


def kernel(xyv, params):
    xyz0 = xyv[:, :2, :]
    l1_xy, l2_xy, l3_xy = _fps_all(xyz0)
    l1s = l1_xy.transpose(0, 2, 1)
    l2s = l2_xy.transpose(0, 2, 1)
    l3s = l3_xy.transpose(0, 2, 1)
    x0s = xyz0.transpose(0, 2, 1)

    def sa(level, xyz, points_nc, nxy_s2, branches):
        featx = jnp.concatenate([points_nc, xyz.transpose(0, 2, 1)], axis=2)
        weights = [[_fold(p) for p in br] for br in params[level]]
        outs = _sa_level(xyz, featx, nxy_s2, branches, weights)
        return jnp.concatenate(outs, axis=2)

    l1_points = sa('sa1', xyz0, xyv.transpose(0, 2, 1), l1s, [(8, 1.0), (32, 9.0)])
    l2_points = sa('sa2', l1_xy, l1_points, l2s, [(8, 4.0), (32, 16.0)])
    l3_points = sa('sa3', l2_xy, l2_points, l3s, [(16, 9.0), (32, 36.0)])

    fp1w = [_fold(p) for p in params['fp1']]
    fp2w = [_fold(p) for p in params['fp2']]
    fp3w = [_fold(p) for p in params['fp3']]
    l2_new = _fp_stage(l2s, l3_xy, l3_points, None, fp1w)
    l1_new = _fp_stage(l1s, l2_xy, l2_new, l1_points, fp2w)
    l0_new = _fp_stage(x0s, l1_xy, l1_new, xyv.transpose(0, 2, 1), fp3w)

    w1, b1 = _fold_conv(params['conv1'], params['bn1'])
    w2, b2 = _fold_conv(params['conv2'], params['bn2'])
    w3, b3 = _fold_conv(params['conv3'], None)
    x = _conv_head(l0_new, w1, b1, w2, b2, w3, b3)
    return (x, l3_points.transpose(0, 2, 1))


# speedup vs baseline: 10.1034x; 1.4806x over previous
import functools

import jax
import jax.numpy as jnp
from jax.experimental import pallas as pl
from jax.experimental.pallas import tpu as pltpu

EPS = 1e-5
NEG = -1e30
_INTERPRET = False

_mm = functools.partial(jnp.dot, preferred_element_type=jnp.float32)


def _fps_body(xy_ref, o1_ref, o2_ref, o3_ref):
    def run(px, py, S):
        Bp, N = px.shape
        iota_n = jax.lax.broadcasted_iota(jnp.int32, (Bp, N), 1)
        iota_s = jax.lax.broadcasted_iota(jnp.int32, (Bp, S), 1)

        def step(i, state):
            dist, far, ax, ay = state
            onehot = (iota_n == far).astype(jnp.float32)
            cx = jnp.sum(px * onehot, axis=1, keepdims=True)
            cy = jnp.sum(py * onehot, axis=1, keepdims=True)
            emit = (iota_s == i).astype(jnp.float32)
            ax = ax + cx * emit
            ay = ay + cy * emit
            d = (px - cx) ** 2 + (py - cy) ** 2
            dist = jnp.minimum(dist, d)
            far = jnp.argmax(dist, axis=1).reshape(Bp, 1).astype(jnp.int32)
            return dist, far, ax, ay

        init = (jnp.full((Bp, N), 1e10, jnp.float32),
                jnp.zeros((Bp, 1), jnp.int32),
                jnp.zeros((Bp, S), jnp.float32),
                jnp.zeros((Bp, S), jnp.float32))
        _, _, ax, ay = jax.lax.fori_loop(0, S, step, init)
        return ax, ay

    px, py = xy_ref[:, 0, :], xy_ref[:, 1, :]
    for o_ref, S in ((o1_ref, 1024), (o2_ref, 512), (o3_ref, 256)):
        px, py = run(px, py, S)
        o_ref[:, 0, :] = px
        o_ref[:, 1, :] = py


def _fps_all(xy):
    B, _, N = xy.shape
    outs = [jax.ShapeDtypeStruct((B, 2, s), jnp.float32) for s in (1024, 512, 256)]
    return pl.pallas_call(
        _fps_body,
        grid=(1,),
        in_specs=[pl.BlockSpec((B, 2, N), lambda b: (0, 0, 0))],
        out_specs=[pl.BlockSpec((B, 2, s), lambda b: (0, 0, 0))
                   for s in (1024, 512, 256)],
        out_shape=outs,
        interpret=_INTERPRET,
    )(xy)


def _make_sa_body(N, T, C, branches):
    def body(x2_ref, featx_ref, nxy_ref, *refs):
        wrefs = refs[:len(branches) * 6]
        orefs = refs[len(branches) * 6:]
        x2 = x2_ref[0]
        c = nxy_ref[0]
        s_dst = jnp.sum(x2 * x2, axis=0, keepdims=True)
        s_src = jnp.sum(c * c, axis=1, keepdims=True)
        sqr = s_src + s_dst - 2.0 * _mm(c, x2)
        featx = featx_ref[0]
        for bi, (K, r2) in enumerate(branches):
            match = sqr <= r2
            rank = match.astype(jnp.int32)
            sh = 1
            while sh < N:
                rank = rank + jnp.concatenate(
                    [jnp.zeros((T, sh), jnp.int32), rank[:, :N - sh]], axis=1)
                sh *= 2
            cnt = rank[:, N - 1:N]
            rank_t = jnp.concatenate([rank] * K, axis=0)
            match_t = jnp.concatenate([match] * K, axis=0)
            kk = jax.lax.broadcasted_iota(jnp.int32, (K * T, 1), 0) // T
            sel = jnp.where(jnp.logical_and(match_t, rank_t == kk + 1),
                            1.0, 0.0)
            g = _mm(sel, featx)
            ct = jnp.concatenate([c] * K, axis=0)
            h = jnp.concatenate([g[:, :C], g[:, C:] - ct], axis=1)
            for li in range(3):
                w = wrefs[bi * 6 + 2 * li][...]
                b = wrefs[bi * 6 + 2 * li + 1][...]
                h = jnp.maximum(_mm(h, w) + b, 0.0)
            cnt_t = jnp.concatenate([cnt] * K, axis=0)
            h = jnp.where(kk < cnt_t, h, NEG)
            o = h[0:T, :]
            for k in range(1, K):
                o = jnp.maximum(o, h[k * T:(k + 1) * T, :])
            orefs[bi][0] = o
    return body


def _sa_level(xyz, featx, nxy, branches, weights, T=32):
    B, _, N = xyz.shape
    S = nxy.shape[1]
    C = featx.shape[2] - 2
    in_specs = [
        pl.BlockSpec((1, 2, N), lambda b, t: (b, 0, 0)),
        pl.BlockSpec((1, N, featx.shape[2]), lambda b, t: (b, 0, 0)),
        pl.BlockSpec((1, T, 2), lambda b, t: (b, t, 0)),
    ]
    args = [xyz, featx, nxy]
    for br in weights:
        for (w, b) in br:
            args += [w, b]
            in_specs += [
                pl.BlockSpec(w.shape, lambda bb, tt: (0, 0)),
                pl.BlockSpec(b.shape, lambda bb, tt: (0, 0)),
            ]
    outs = [jax.ShapeDtypeStruct((B, S, br[-1][0].shape[1]), jnp.float32)
            for br in weights]
    out_specs = [pl.BlockSpec((1, T, o.shape[2]), lambda b, t: (b, t, 0))
                 for o in outs]
    return pl.pallas_call(
        _make_sa_body(N, T, C, branches),
        grid=(B, S // T),
        in_specs=in_specs,
        out_specs=out_specs,
        out_shape=outs,
        interpret=_INTERPRET,
    )(*args)


def _make_fp_body(N2, T, has_p1, nlayers):
    def body(*refs):
        x1_ref, x2_ref, p2_ref = refs[0], refs[1], refs[2]
        i = 3
        p1_ref = None
        if has_p1:
            p1_ref = refs[i]
            i += 1
        wrefs = refs[i:i + 2 * nlayers]
        out_ref = refs[i + 2 * nlayers]
        c = x1_ref[0]
        x2 = x2_ref[0]
        s2 = jnp.sum(x2 * x2, axis=0, keepdims=True)
        s1 = jnp.sum(c * c, axis=1, keepdims=True)
        d = s1 + s2 - 2.0 * _mm(c, x2)
        iota_n = jax.lax.broadcasted_iota(jnp.int32, (T, N2), 1)
        dd = d
        dmins, idxs = [], []
        for _ in range(3):
            dmins.append(jnp.min(dd, axis=1, keepdims=True))
            im = jnp.argmin(dd, axis=1).reshape(T, 1).astype(jnp.int32)
            idxs.append(im)
            dd = jnp.where(iota_n == im, 1e30, dd)
        recips = [1.0 / (dm + 1e-8) for dm in dmins]
        norm = recips[0] + recips[1] + recips[2]
        wmat = jnp.zeros((T, N2), jnp.float32)
        for k in range(3):
            wmat = wmat + jnp.where(iota_n == idxs[k], 1.0, 0.0) * (recips[k] / norm)
        h = _mm(wmat, p2_ref[0])
        if has_p1:
            h = jnp.concatenate([p1_ref[0], h], axis=1)
        for li in range(nlayers):
            h = jnp.maximum(_mm(h, wrefs[2 * li][...]) + wrefs[2 * li + 1][...], 0.0)
        out_ref[0] = h
    return body


def _fp_stage(x1s, xyz2, p2, p1, weights, T=256):
    B, N1, _ = x1s.shape
    N2 = xyz2.shape[2]
    in_specs = [
        pl.BlockSpec((1, T, 2), lambda b, t: (b, t, 0)),
        pl.BlockSpec((1, 2, N2), lambda b, t: (b, 0, 0)),
        pl.BlockSpec((1, N2, p2.shape[2]), lambda b, t: (b, 0, 0)),
    ]
    args = [x1s, xyz2, p2]
    if p1 is not None:
        in_specs.append(pl.BlockSpec((1, T, p1.shape[2]), lambda b, t: (b, t, 0)))
        args.append(p1)
    for (w, b) in weights:
        args += [w, b]
        in_specs += [
            pl.BlockSpec(w.shape, lambda bb, tt: (0, 0)),
            pl.BlockSpec(b.shape, lambda bb, tt: (0, 0)),
        ]
    O = weights[-1][0].shape[1]
    out = jax.ShapeDtypeStruct((B, N1, O), jnp.float32)
    return pl.pallas_call(
        _make_fp_body(N2, T, p1 is not None, len(weights)),
        grid=(B, N1 // T),
        in_specs=in_specs,
        out_specs=pl.BlockSpec((1, T, O), lambda b, t: (b, t, 0)),
        out_shape=out,
        interpret=_INTERPRET,
    )(*args)


def _conv_body(x_ref, w1_ref, b1_ref, w2_ref, b2_ref, w3_ref, b3_ref,
               out_ref, xp_ref, acc_ref):
    def stage(xin, w_ref, b, front, L, Lout, relu, G):
        ngroups = w_ref.shape[0]
        O = w_ref.shape[2]
        xp_ref[...] = jnp.zeros(xp_ref.shape, jnp.float32)
        xp_ref[pl.ds(front, L), :] = xin
        acc_ref[:Lout, :O] = jnp.zeros((Lout, O), jnp.float32)

        def tau(t, _):
            base = t * G
            xs = jnp.concatenate(
                [xp_ref[pl.ds(base + i, Lout), :] for i in range(G)], axis=1)
            acc_ref[:Lout, :O] += _mm(xs, w_ref[t])
            return 0

        jax.lax.fori_loop(0, ngroups, tau, 0)
        h = acc_ref[:Lout, :O] + b
        return jnp.maximum(h, 0.0) if relu else h

    x = x_ref[0]
    h1 = stage(x, w1_ref, b1_ref[...], 128, 2048, 2048, True, 8)
    h2 = stage(h1, w2_ref, b2_ref[...], 63, 2048, 2047, True, 8)
    h3 = stage(h2, w3_ref, b3_ref[...], 3, 2047, 2048, False, 1)
    m = jnp.max(h3, axis=1, keepdims=True)
    z = h3 - m
    lse = jnp.log(jnp.sum(jnp.exp(z), axis=1, keepdims=True))
    out_ref[0] = z - lse


def _conv_head(x, w1, b1, w2, b2, w3, b3):
    B = x.shape[0]
    wspecs = []
    for w, b in ((w1, b1), (w2, b2), (w3, b3)):
        wspecs += [pl.BlockSpec(w.shape, lambda bb: (0, 0, 0)),
                   pl.BlockSpec(b.shape, lambda bb: (0, 0))]
    return pl.pallas_call(
        _conv_body,
        grid=(B,),
        in_specs=[pl.BlockSpec((1, 2048, 128), lambda b: (b, 0, 0))] + wspecs,
        out_specs=pl.BlockSpec((1, 2048, 2), lambda b: (b, 0, 0)),
        out_shape=jax.ShapeDtypeStruct((B, 2048, 2), jnp.float32),
        scratch_shapes=[pltpu.VMEM((2304, 128), jnp.float32),
                        pltpu.VMEM((2048, 128), jnp.float32)],
        interpret=_INTERPRET,
    )(x, w1, b1, w2, b2, w3, b3)


def _fold(p):
    s = p['gamma'] / jnp.sqrt(1.0 + EPS)
    return (p['W'] * s[:, None]).T, (p['b'] * s + p['beta'])[None, :]


def _fold_conv(p, bn, G):
    if bn is None:
        w, b = p['W'], p['b']
    else:
        s = bn['gamma'] / jnp.sqrt(1.0 + EPS)
        w = p['W'] * s[:, None, None]
        b = p['b'] * s + bn['beta']
    w = w.transpose(2, 1, 0)
    k, I, O = w.shape
    return w.reshape(k // G, G * I, O), b[None, :]


def kernel(xyv, params):
    xyz0 = xyv[:, :2, :]
    l1_xy, l2_xy, l3_xy = _fps_all(xyz0)
    l1s = l1_xy.transpose(0, 2, 1)
    l2s = l2_xy.transpose(0, 2, 1)
    l3s = l3_xy.transpose(0, 2, 1)
    x0s = xyz0.transpose(0, 2, 1)

    def sa(level, xyz, points_nc, nxy_s2, branches):
        featx = jnp.concatenate([points_nc, xyz.transpose(0, 2, 1)], axis=2)
        weights = [[_fold(p) for p in br] for br in params[level]]
        outs = _sa_level(xyz, featx, nxy_s2, branches, weights)
        return jnp.concatenate(outs, axis=2)

    l1_points = sa('sa1', xyz0, xyv.transpose(0, 2, 1), l1s, [(8, 1.0), (32, 9.0)])
    l2_points = sa('sa2', l1_xy, l1_points, l2s, [(8, 4.0), (32, 16.0)])
    l3_points = sa('sa3', l2_xy, l2_points, l3s, [(16, 9.0), (32, 36.0)])

    fp1w = [_fold(p) for p in params['fp1']]
    fp2w = [_fold(p) for p in params['fp2']]
    fp3w = [_fold(p) for p in params['fp3']]
    l2_new = _fp_stage(l2s, l3_xy, l3_points, None, fp1w)
    l1_new = _fp_stage(l1s, l2_xy, l2_new, l1_points, fp2w)
    l0_new = _fp_stage(x0s, l1_xy, l1_new, xyv.transpose(0, 2, 1), fp3w)

    w1, b1 = _fold_conv(params['conv1'], params['bn1'], 8)
    w2, b2 = _fold_conv(params['conv2'], params['bn2'], 8)
    w3, b3 = _fold_conv(params['conv3'], None, 1)
    x = _conv_head(l0_new, w1, b1, w2, b2, w3, b3)
    return (x, l3_points.transpose(0, 2, 1))
